# baseline (speedup 1.0000x reference)
import jax
import jax.numpy as jnp
from jax.experimental import pallas as pl

_N = 65536
_M = 1024
_B = 512


def _tile_kernel(dd_ref, dt_ref, cd_ref, ct_ref, asn_ref, w_ref):
    dd = dd_ref[:]
    dt = dt_ref[:]
    cd = cd_ref[:]
    ct = ct_ref[:]

    depth_diff = (dd - cd) ** 2
    light_penalty = 1.0 - jnp.exp(-0.045 * cd)
    time_diff = (dt - ct) ** 2
    time_diff = time_diff / 3600.0
    cost = 1.0 * depth_diff + 0.5 * light_penalty + 0.3 * time_diff

    min_cost = jnp.min(cost, axis=1, keepdims=True)
    ids = jax.lax.broadcasted_iota(jnp.int32, cost.shape, 1)
    min_j = jnp.min(jnp.where(cost == min_cost, ids, _M), axis=1,
                    keepdims=True)

    valid = min_cost < 625.0
    asn_ref[:] = jnp.where(valid, min_j, -1)
    w_ref[:] = jnp.where(valid, 1.0 / (1.0 + jnp.sqrt(min_cost)), 0.0)


def kernel(detection_depths, camera_depths, detection_times, camera_times):
    n = detection_depths.shape[0]
    m = camera_depths.shape[0]
    dd = detection_depths.reshape(n, 1)
    dt = detection_times.reshape(n, 1)
    cd = camera_depths.reshape(1, m)
    ct = camera_times.reshape(1, m)

    grid = (n // _B,)
    asn, w = pl.pallas_call(
        _tile_kernel,
        grid=grid,
        in_specs=[
            pl.BlockSpec((_B, 1), lambda i: (i, 0)),
            pl.BlockSpec((_B, 1), lambda i: (i, 0)),
            pl.BlockSpec((1, m), lambda i: (0, 0)),
            pl.BlockSpec((1, m), lambda i: (0, 0)),
        ],
        out_specs=[
            pl.BlockSpec((_B, 1), lambda i: (i, 0)),
            pl.BlockSpec((_B, 1), lambda i: (i, 0)),
        ],
        out_shape=[
            jax.ShapeDtypeStruct((n, 1), jnp.int32),
            jax.ShapeDtypeStruct((n, 1), jnp.float32),
        ],
    )(dd, dt, cd, ct)

    assignments = asn.reshape(n).astype(jnp.int64)
    weights = w.reshape(n)
    return assignments, weights


# speedup vs baseline: 1.5246x; 1.5246x over previous
import jax
import jax.numpy as jnp
from jax.experimental import pallas as pl

_M = 1024
_B = 512
_TS = (0.3 / 3600.0) ** 0.5


def _tile_kernel(dd_ref, sdt_ref, cd_ref, sct_ref, hlp_ref, asn_ref, w_ref):
    dd = dd_ref[:]
    sdt = sdt_ref[:]
    cd = cd_ref[:]
    sct = sct_ref[:]
    hlp = hlp_ref[:]

    d1 = dd - cd
    t1 = sdt - sct
    cost = (d1 * d1 + hlp) + t1 * t1

    min_cost = jnp.min(cost, axis=0, keepdims=True)
    ids = jax.lax.broadcasted_iota(jnp.int32, (_M, 1), 0)
    min_j = jnp.min(jnp.where(cost == min_cost, ids, _M), axis=0,
                    keepdims=True)

    valid = min_cost < 625.0
    asn_ref[:] = jnp.where(valid, min_j, -1)
    w_ref[:] = jnp.where(valid, 1.0 / (1.0 + jnp.sqrt(min_cost)), 0.0)


def kernel(detection_depths, camera_depths, detection_times, camera_times):
    n = detection_depths.shape[0]
    m = camera_depths.shape[0]
    dd = detection_depths.reshape(1, n)
    sdt = (detection_times * _TS).reshape(1, n)
    cd = camera_depths.reshape(m, 1)
    sct = (camera_times * _TS).reshape(m, 1)
    hlp = (0.5 * (1.0 - jnp.exp(-0.045 * camera_depths))).reshape(m, 1)

    grid = (n // _B,)
    asn, w = pl.pallas_call(
        _tile_kernel,
        grid=grid,
        in_specs=[
            pl.BlockSpec((1, _B), lambda i: (0, i)),
            pl.BlockSpec((1, _B), lambda i: (0, i)),
            pl.BlockSpec((m, 1), lambda i: (0, 0)),
            pl.BlockSpec((m, 1), lambda i: (0, 0)),
            pl.BlockSpec((m, 1), lambda i: (0, 0)),
        ],
        out_specs=[
            pl.BlockSpec((1, _B), lambda i: (0, i)),
            pl.BlockSpec((1, _B), lambda i: (0, i)),
        ],
        out_shape=[
            jax.ShapeDtypeStruct((1, n), jnp.int32),
            jax.ShapeDtypeStruct((1, n), jnp.float32),
        ],
    )(dd, sdt, cd, sct, hlp)

    assignments = asn.reshape(n).astype(jnp.int64)
    weights = w.reshape(n)
    return assignments, weights
